# baseline (device time: 51906 ns/iter reference)
import jax
import jax.numpy as jnp
from jax import lax
from jax.experimental import pallas as pl
from jax.experimental.pallas import tpu as pltpu

P = 8
B = 64
D = 512
L = 3

_CompilerParams = getattr(pltpu, "CompilerParams", None) or pltpu.TPUCompilerParams


def kernel(x, Win0, Wout0, Win1, Wout1, Win2, Wout2):
    def body(x_ref, win0_ref, wout0_ref, win1_ref, wout1_ref, win2_ref,
             wout2_ref, out_ref, gather_buf, partial_buf, rs_buf,
             ag_send, ag_recv, rs_send, rs_recv):
        my = lax.axis_index("i")

        barrier = pltpu.get_barrier_semaphore()
        for d in range(1, P):
            pl.semaphore_signal(
                barrier, inc=1,
                device_id=((my + d) % P,),
                device_id_type=pl.DeviceIdType.MESH,
            )
        pl.semaphore_wait(barrier, P - 1)

        def exchange(buf, sem_send, sem_recv, l, src_rows_of):
            sends = []
            for d in range(1, P):
                tgt = (my + d) % P
                rdma = pltpu.make_async_remote_copy(
                    src_ref=src_rows_of(l, tgt),
                    dst_ref=buf.at[l, pl.ds(my * B, B), :],
                    send_sem=sem_send.at[l, tgt],
                    recv_sem=sem_recv.at[l, my],
                    device_id=(tgt,),
                    device_id_type=pl.DeviceIdType.MESH,
                )
                rdma.start()
                sends.append(rdma)
            for d in range(1, P):
                src = (my + P - d) % P
                recv = pltpu.make_async_remote_copy(
                    src_ref=buf.at[l, pl.ds(src * B, B), :],
                    dst_ref=buf.at[l, pl.ds(src * B, B), :],
                    send_sem=sem_send.at[l, src],
                    recv_sem=sem_recv.at[l, src],
                    device_id=(src,),
                    device_id_type=pl.DeviceIdType.MESH,
                )
                recv.wait_recv()
            for rdma in sends:
                rdma.wait_send()

        def ag_round(l):
            exchange(
                gather_buf, ag_send, ag_recv, l,
                lambda l, tgt: gather_buf.at[l, pl.ds(my * B, B), :],
            )

        def rs_round(l):
            rs_buf[l, pl.ds(my * B, B), :] = partial_buf[l, pl.ds(my * B, B), :]
            exchange(
                rs_buf, rs_send, rs_recv, l,
                lambda l, tgt: partial_buf.at[l, pl.ds(tgt * B, B), :],
            )

        wins = [win0_ref, win1_ref, win2_ref]
        wouts = [wout0_ref, wout1_ref, wout2_ref]

        gather_buf[0, pl.ds(my * B, B), :] = x_ref[...].astype(jnp.bfloat16)
        ag_round(0)

        for l in range(L):
            x_full = gather_buf[l, :, :]
            h = jnp.dot(x_full, wins[l][...].astype(jnp.bfloat16),
                        preferred_element_type=jnp.float32)
            h = jnp.maximum(h, 0.0).astype(jnp.bfloat16)
            part = jnp.dot(h, wouts[l][...].astype(jnp.bfloat16),
                           preferred_element_type=jnp.float32)
            partial_buf[l, :, :] = part.astype(jnp.bfloat16)

            rs_round(l)
            acc = jnp.sum(
                rs_buf[l, :, :].astype(jnp.float32).reshape(P, B, D), axis=0
            )
            if l < L - 1:
                gather_buf[l + 1, pl.ds(my * B, B), :] = acc.astype(jnp.bfloat16)
                ag_round(l + 1)
            else:
                out_ref[...] = acc

    return pl.pallas_call(
        body,
        out_shape=jax.ShapeDtypeStruct((B, D), jnp.float32),
        in_specs=[pl.BlockSpec(memory_space=pltpu.VMEM)] * 7,
        out_specs=pl.BlockSpec(memory_space=pltpu.VMEM),
        scratch_shapes=[
            pltpu.VMEM((L, P * B, D), jnp.bfloat16),
            pltpu.VMEM((L, P * B, D), jnp.bfloat16),
            pltpu.VMEM((L, P * B, D), jnp.bfloat16),
            pltpu.SemaphoreType.DMA((L, P)),
            pltpu.SemaphoreType.DMA((L, P)),
            pltpu.SemaphoreType.DMA((L, P)),
            pltpu.SemaphoreType.DMA((L, P)),
        ],
        compiler_params=_CompilerParams(collective_id=0),
    )(x, Win0, Wout0, Win1, Wout1, Win2, Wout2)


# device time: 50877 ns/iter; 1.0202x vs baseline; 1.0202x over previous
import jax
import jax.numpy as jnp
from jax import lax
from jax.experimental import pallas as pl
from jax.experimental.pallas import tpu as pltpu

P = 8
B = 64
D = 512
L = 3

_CompilerParams = getattr(pltpu, "CompilerParams", None) or pltpu.TPUCompilerParams


def kernel(x, Win0, Wout0, Win1, Wout1, Win2, Wout2):
    def body(x_ref, win0_ref, wout0_ref, win1_ref, wout1_ref, win2_ref,
             wout2_ref, out_ref, gather_buf, partial_buf, rs_buf,
             winb, woutb, ag_send, ag_recv, rs_send, rs_recv):
        my = lax.axis_index("i")
        bf16 = jnp.bfloat16

        barrier = pltpu.get_barrier_semaphore()
        for d in range(1, P):
            pl.semaphore_signal(
                barrier, inc=1,
                device_id=((my + d) % P,),
                device_id_type=pl.DeviceIdType.MESH,
            )
        pl.semaphore_wait(barrier, P - 1)

        def block(buf, l, pos):
            return buf.at[l, pl.ds(pos * B, B), :]

        def start_ag(l):
            sends = []
            for d in range(1, P):
                tgt = (my + d) % P
                rdma = pltpu.make_async_remote_copy(
                    src_ref=block(gather_buf, l, my),
                    dst_ref=block(gather_buf, l, my),
                    send_sem=ag_send.at[l, tgt],
                    recv_sem=ag_recv.at[l, my],
                    device_id=(tgt,),
                    device_id_type=pl.DeviceIdType.MESH,
                )
                rdma.start()
                sends.append(rdma)
            return sends

        def wait_recv(buf, sem, l, src):
            pltpu.make_async_remote_copy(
                src_ref=block(buf, l, src),
                dst_ref=block(buf, l, src),
                send_sem=sem.at[l, src],
                recv_sem=sem.at[l, src],
                device_id=(src,),
                device_id_type=pl.DeviceIdType.MESH,
            ).wait_recv()

        wins = [win0_ref, win1_ref, win2_ref]
        wouts = [wout0_ref, wout1_ref, wout2_ref]
        deferred_sends = []

        gather_buf[0, pl.ds(my * B, B), :] = x_ref[...].astype(bf16)
        deferred_sends += start_ag(0)

        for l in range(L):
            winb[...] = wins[l][...].astype(bf16)
            woutb[...] = wouts[l][...].astype(bf16)

            def pblock(xb):
                h = jnp.dot(xb, winb[...], preferred_element_type=jnp.float32)
                h = jnp.maximum(h, 0.0).astype(bf16)
                p = jnp.dot(h, woutb[...], preferred_element_type=jnp.float32)
                return p.astype(bf16)

            rs_buf[l, pl.ds(my * B, B), :] = pblock(
                gather_buf[l, pl.ds(my * B, B), :])

            for d in range(1, P):
                s = (my + d) % P
                wait_recv(gather_buf, ag_recv, l, s)
                partial_buf[l, pl.ds(s * B, B), :] = pblock(
                    gather_buf[l, pl.ds(s * B, B), :])
                rdma = pltpu.make_async_remote_copy(
                    src_ref=block(partial_buf, l, s),
                    dst_ref=block(rs_buf, l, my),
                    send_sem=rs_send.at[l, s],
                    recv_sem=rs_recv.at[l, my],
                    device_id=(s,),
                    device_id_type=pl.DeviceIdType.MESH,
                )
                rdma.start()
                deferred_sends.append(rdma)

            for d in range(1, P):
                s = (my + P - d) % P
                wait_recv(rs_buf, rs_recv, l, s)
            acc = jnp.sum(
                rs_buf[l, :, :].astype(jnp.float32).reshape(P, B, D), axis=0
            )
            if l < L - 1:
                gather_buf[l + 1, pl.ds(my * B, B), :] = acc.astype(bf16)
                deferred_sends += start_ag(l + 1)
            else:
                out_ref[...] = acc

        for rdma in deferred_sends:
            rdma.wait_send()

    return pl.pallas_call(
        body,
        out_shape=jax.ShapeDtypeStruct((B, D), jnp.float32),
        in_specs=[pl.BlockSpec(memory_space=pltpu.VMEM)] * 7,
        out_specs=pl.BlockSpec(memory_space=pltpu.VMEM),
        scratch_shapes=[
            pltpu.VMEM((L, P * B, D), jnp.bfloat16),
            pltpu.VMEM((L, P * B, D), jnp.bfloat16),
            pltpu.VMEM((L, P * B, D), jnp.bfloat16),
            pltpu.VMEM((D, 2 * D), jnp.bfloat16),
            pltpu.VMEM((2 * D, D), jnp.bfloat16),
            pltpu.SemaphoreType.DMA((L, P)),
            pltpu.SemaphoreType.DMA((L, P)),
            pltpu.SemaphoreType.DMA((L, P)),
            pltpu.SemaphoreType.DMA((L, P)),
        ],
        compiler_params=_CompilerParams(collective_id=0),
    )(x, Win0, Wout0, Win1, Wout1, Win2, Wout2)
